# baseline (speedup 1.0000x reference)
import functools

import jax
import jax.numpy as jnp
from jax import lax
from jax.experimental import pallas as pl
from jax.experimental.pallas import tpu as pltpu
from jax.experimental.pallas import tpu_sc as plsc

BATCH = 16384
EMBED_DIM = 64
LANES = 16
NUM_CORES = 2
NUM_SUBCORES = 16
NUM_WORKERS = NUM_CORES * NUM_SUBCORES
B_PER_W = BATCH // NUM_WORKERS
BLOCKS_PER_W = B_PER_W // LANES

_MESH = dict(core_axis_name="c", subcore_axis_name="s",
             num_cores=NUM_CORES, num_subcores=NUM_SUBCORES)
_PARAMS = pltpu.CompilerParams(needs_layout_passes=False,
                               use_tc_tiling_on_sc=False,
                               skip_device_barrier=True)


def _worker_base():
    wid = lax.axis_index("s") * NUM_CORES + lax.axis_index("c")
    return wid * B_PER_W


def _prep_body(uid_hbm, iid_hbm, it_hbm, ub_hbm, ib_hbm, gb_hbm,
               irows_hbm, pbias_hbm,
               uidx, iidx, irows, ubias, ibias, gbv, pbv, sem, semb):
    base = _worker_base()
    pltpu.sync_copy(uid_hbm.at[pl.ds(base, B_PER_W)], uidx)
    pltpu.sync_copy(iid_hbm.at[pl.ds(base, B_PER_W)], iidx)

    ci = pltpu.async_copy(it_hbm.at[iidx], irows, sem)
    cub = pltpu.async_copy(ub_hbm.at[uidx], ubias, semb)
    cib = pltpu.async_copy(ib_hbm.at[iidx], ibias, semb)
    pltpu.sync_copy(gb_hbm, gbv)
    cub.wait()
    cib.wait()

    gb = gbv[...]

    def bias_step(i, _):
        sl = pl.ds(i * LANES, LANES)
        pbv[sl] = ubias[sl] + ibias[sl] + gb
        return 0

    lax.fori_loop(0, BLOCKS_PER_W, bias_step, 0)
    ci.wait()

    pltpu.sync_copy(pbv, pbias_hbm.at[pl.ds(base, B_PER_W)])
    pltpu.sync_copy(irows, irows_hbm.at[pl.ds(base, B_PER_W)])


HALF_USERS = 500000


def _dot_body(uid_hbm, ut0_hbm, ut1_hbm, irows_hbm, pbias_hbm,
              out_hbm,
              uidx, uidx0, uidx1, urows0, urows1, irows, pbv, outv, sem):
    base = _worker_base()
    pltpu.sync_copy(uid_hbm.at[pl.ds(base, B_PER_W)], uidx)

    def split_step(i, _):
        sl = pl.ds(i * LANES, LANES)
        v = uidx[sl]
        uidx0[sl] = jnp.minimum(v, HALF_USERS - 1)
        uidx1[sl] = jnp.maximum(v - HALF_USERS, 0)
        return 0

    lax.fori_loop(0, BLOCKS_PER_W, split_step, 0)

    cu0 = pltpu.async_copy(ut0_hbm.at[uidx0], urows0, sem)
    cu1 = pltpu.async_copy(ut1_hbm.at[uidx1], urows1, sem)
    ci = pltpu.async_copy(irows_hbm.at[pl.ds(base, B_PER_W)], irows, sem)
    cp = pltpu.async_copy(pbias_hbm.at[pl.ds(base, B_PER_W)], pbv, sem)
    cu0.wait()
    cu1.wait()
    ci.wait()
    cp.wait()

    lane = lax.iota(jnp.int32, LANES)

    def block(b, _):
        rowbase = b * LANES
        rows = rowbase + lane
        hi = uidx[pl.ds(rowbase, LANES)] >= HALF_USERS
        acc = pbv[pl.ds(rowbase, LANES)]
        for d in range(EMBED_DIM):
            col = jnp.full((LANES,), d, jnp.int32)
            u0 = plsc.load_gather(urows0, [rows, col])
            u1 = plsc.load_gather(urows1, [rows, col])
            uv = jnp.where(hi, u1, u0)
            iv = plsc.load_gather(irows, [rows, col])
            acc = acc + uv * iv
        outv[pl.ds(rowbase, LANES)] = acc
        return 0

    lax.fori_loop(0, BLOCKS_PER_W, block, 0)

    pltpu.sync_copy(outv, out_hbm.at[pl.ds(base, B_PER_W)])


@jax.jit
def kernel(user_ids, item_ids, user_table, item_table, user_bias_table,
           item_bias_table, global_bias):
    uid32 = user_ids.astype(jnp.int32)
    iid32 = item_ids.astype(jnp.int32)
    gb16 = jnp.broadcast_to(global_bias, (LANES,))

    prep = pl.kernel(
        _prep_body,
        out_type=(jax.ShapeDtypeStruct((BATCH, EMBED_DIM), jnp.float32),
                  jax.ShapeDtypeStruct((BATCH,), jnp.float32)),
        mesh=plsc.VectorSubcoreMesh(**_MESH),
        scratch_types=[
            pltpu.VMEM((B_PER_W,), jnp.int32),
            pltpu.VMEM((B_PER_W,), jnp.int32),
            pltpu.VMEM((B_PER_W, EMBED_DIM), jnp.float32),
            pltpu.VMEM((B_PER_W,), jnp.float32),
            pltpu.VMEM((B_PER_W,), jnp.float32),
            pltpu.VMEM((LANES,), jnp.float32),
            pltpu.VMEM((B_PER_W,), jnp.float32),
            pltpu.SemaphoreType.DMA,
            pltpu.SemaphoreType.DMA,
        ],
        compiler_params=_PARAMS,
    )
    irows, pbias = prep(uid32, iid32, item_table,
                        user_bias_table.reshape(-1),
                        item_bias_table.reshape(-1), gb16)

    dot = pl.kernel(
        _dot_body,
        out_type=jax.ShapeDtypeStruct((BATCH,), jnp.float32),
        mesh=plsc.VectorSubcoreMesh(**_MESH),
        scratch_types=[
            pltpu.VMEM((B_PER_W,), jnp.int32),
            pltpu.VMEM((B_PER_W,), jnp.int32),
            pltpu.VMEM((B_PER_W,), jnp.int32),
            pltpu.VMEM((B_PER_W, EMBED_DIM), jnp.float32),
            pltpu.VMEM((B_PER_W, EMBED_DIM), jnp.float32),
            pltpu.VMEM((B_PER_W, EMBED_DIM), jnp.float32),
            pltpu.VMEM((B_PER_W,), jnp.float32),
            pltpu.VMEM((B_PER_W,), jnp.float32),
            pltpu.SemaphoreType.DMA,
        ],
        compiler_params=_PARAMS,
    )
    return dot(uid32, user_table[:HALF_USERS], user_table[HALF_USERS:],
               irows, pbias)


# speedup vs baseline: 1.7379x; 1.7379x over previous
import functools

import jax
import jax.numpy as jnp
from jax import lax
from jax.experimental import pallas as pl
from jax.experimental.pallas import tpu as pltpu
from jax.experimental.pallas import tpu_sc as plsc

BATCH = 16384
EMBED_DIM = 64
LANES = 16
NUM_CORES = 2
NUM_SUBCORES = 16
NUM_WORKERS = NUM_CORES * NUM_SUBCORES
B_PER_W = BATCH // NUM_WORKERS
BLOCKS_PER_W = B_PER_W // LANES

_MESH = dict(core_axis_name="c", subcore_axis_name="s",
             num_cores=NUM_CORES, num_subcores=NUM_SUBCORES)
_PARAMS = pltpu.CompilerParams(needs_layout_passes=False,
                               use_tc_tiling_on_sc=False)


def _worker_base():
    wid = lax.axis_index("s") * NUM_CORES + lax.axis_index("c")
    return wid * B_PER_W


def _prep_body(iid_hbm, it_hbm, ubg_hbm, ibg_hbm, gb_hbm,
               irows_hbm, pbias_hbm,
               iidx, irows, ubias, ibias, gbv, pbv, sem, semb):
    base = _worker_base()
    pltpu.sync_copy(iid_hbm.at[pl.ds(base, B_PER_W)], iidx)

    ci = pltpu.async_copy(it_hbm.at[iidx], irows, sem)
    cub = pltpu.async_copy(ubg_hbm.at[pl.ds(base, B_PER_W)], ubias, semb)
    cib = pltpu.async_copy(ibg_hbm.at[pl.ds(base, B_PER_W)], ibias, semb)
    pltpu.sync_copy(gb_hbm, gbv)
    cub.wait()
    cib.wait()

    gb = gbv[...]

    def bias_step(i, _):
        sl = pl.ds(i * LANES, LANES)
        pbv[sl] = ubias[sl] + ibias[sl] + gb
        return 0

    lax.fori_loop(0, BLOCKS_PER_W, bias_step, 0)
    ci.wait()

    pltpu.sync_copy(pbv, pbias_hbm.at[pl.ds(base, B_PER_W)])
    pltpu.sync_copy(irows, irows_hbm.at[pl.ds(base, B_PER_W)])


def _dot_body(uid_hbm, ut_hbm, irows_hbm, pbias_hbm,
              out_hbm,
              uidx, urows, irows, pbv, outv, sem):
    base = _worker_base()
    pltpu.sync_copy(uid_hbm.at[pl.ds(base, B_PER_W)], uidx)

    cu = pltpu.async_copy(ut_hbm.at[uidx], urows, sem)
    ci = pltpu.async_copy(irows_hbm.at[pl.ds(base, B_PER_W)], irows, sem)
    cp = pltpu.async_copy(pbias_hbm.at[pl.ds(base, B_PER_W)], pbv, sem)
    cu.wait()
    ci.wait()
    cp.wait()

    lane = lax.iota(jnp.int32, LANES)

    def block(b, _):
        rowbase = b * LANES
        rows = rowbase + lane
        acc = pbv[pl.ds(rowbase, LANES)]
        for d in range(EMBED_DIM):
            col = jnp.full((LANES,), d, jnp.int32)
            uv = plsc.load_gather(urows, [rows, col])
            iv = plsc.load_gather(irows, [rows, col])
            acc = acc + uv * iv
        outv[pl.ds(rowbase, LANES)] = acc
        return 0

    lax.fori_loop(0, BLOCKS_PER_W, block, 0)

    pltpu.sync_copy(outv, out_hbm.at[pl.ds(base, B_PER_W)])


@jax.jit
def kernel(user_ids, item_ids, user_table, item_table, user_bias_table,
           item_bias_table, global_bias):
    uid32 = user_ids.astype(jnp.int32)
    iid32 = item_ids.astype(jnp.int32)
    gb16 = jnp.broadcast_to(global_bias, (LANES,))
    ubg = jnp.take(user_bias_table, uid32, axis=0).reshape(BATCH)
    ibg = jnp.take(item_bias_table, iid32, axis=0).reshape(BATCH)

    prep = pl.kernel(
        _prep_body,
        out_type=(jax.ShapeDtypeStruct((BATCH, EMBED_DIM), jnp.float32),
                  jax.ShapeDtypeStruct((BATCH,), jnp.float32)),
        mesh=plsc.VectorSubcoreMesh(**_MESH),
        scratch_types=[
            pltpu.VMEM((B_PER_W,), jnp.int32),
            pltpu.VMEM((B_PER_W, EMBED_DIM), jnp.float32),
            pltpu.VMEM((B_PER_W,), jnp.float32),
            pltpu.VMEM((B_PER_W,), jnp.float32),
            pltpu.VMEM((LANES,), jnp.float32),
            pltpu.VMEM((B_PER_W,), jnp.float32),
            pltpu.SemaphoreType.DMA,
            pltpu.SemaphoreType.DMA,
        ],
        compiler_params=_PARAMS,
    )
    irows, pbias = prep(iid32, item_table, ubg, ibg, gb16)

    dot = pl.kernel(
        _dot_body,
        out_type=jax.ShapeDtypeStruct((BATCH,), jnp.float32),
        mesh=plsc.VectorSubcoreMesh(**_MESH),
        scratch_types=[
            pltpu.VMEM((B_PER_W,), jnp.int32),
            pltpu.VMEM((B_PER_W, EMBED_DIM), jnp.float32),
            pltpu.VMEM((B_PER_W, EMBED_DIM), jnp.float32),
            pltpu.VMEM((B_PER_W,), jnp.float32),
            pltpu.VMEM((B_PER_W,), jnp.float32),
            pltpu.SemaphoreType.DMA,
        ],
        compiler_params=_PARAMS,
    )
    return dot(uid32, user_table, irows, pbias)


# speedup vs baseline: 3.1734x; 1.8260x over previous
import functools

import jax
import jax.numpy as jnp
from jax import lax
from jax.experimental import pallas as pl
from jax.experimental.pallas import tpu as pltpu
from jax.experimental.pallas import tpu_sc as plsc

BATCH = 16384
EMBED_DIM = 64
SUBROWS = 8
LANES = 16
NUM_CORES = 2
NUM_SUBCORES = 16
NUM_WORKERS = NUM_CORES * NUM_SUBCORES
B_PER_W = BATCH // NUM_WORKERS
CHUNK = LANES
CHUNKS_PER_W = B_PER_W // CHUNK

_PARAMS = pltpu.CompilerParams(needs_layout_passes=False,
                               use_tc_tiling_on_sc=True)


def _mf_body(uid_hbm, iid_hbm, ut_hbm, it_hbm, ubg_hbm, ibg_hbm, gb_hbm,
             out_hbm,
             uidx, iidx, ubuf, ibuf, ubias, ibias, gbv,
             outv, sem, semb):
    wid = lax.axis_index("s") * NUM_CORES + lax.axis_index("c")
    base = wid * B_PER_W

    pltpu.sync_copy(uid_hbm.at[pl.ds(base, B_PER_W)], uidx)
    pltpu.sync_copy(iid_hbm.at[pl.ds(base, B_PER_W)], iidx)

    cub = pltpu.async_copy(ubg_hbm.at[pl.ds(base, B_PER_W)], ubias, semb)
    cib = pltpu.async_copy(ibg_hbm.at[pl.ds(base, B_PER_W)], ibias, semb)
    pltpu.sync_copy(gb_hbm, gbv)
    cub.wait()
    cib.wait()

    gb = gbv[...]
    lane = lax.iota(jnp.int32, LANES)
    seven = jnp.full((LANES,), 7, jnp.int32)

    def chunk_step(c, _):
        cbase = c * CHUNK
        sl = pl.ds(cbase, LANES)
        idu = uidx[sl]
        idi = iidx[sl]
        copies = []
        for k in range(CHUNK):
            mk = lane == k
            su = jnp.max(jnp.where(mk, idu, 0), axis=0)
            si = jnp.max(jnp.where(mk, idi, 0), axis=0)
            tu = lax.shift_right_logical(su, 3)
            ti = lax.shift_right_logical(si, 3)
            copies.append(pltpu.async_copy(ut_hbm.at[tu], ubuf.at[k], sem))
            copies.append(pltpu.async_copy(it_hbm.at[ti], ibuf.at[k], sem))
        for cp in copies:
            cp.wait()

        usub = lax.bitwise_and(idu, seven)
        isub = lax.bitwise_and(idi, seven)
        acc = gb + ubias[sl] + ibias[sl]
        for d in range(EMBED_DIM):
            col = jnp.full((LANES,), d, jnp.int32)
            uv = plsc.load_gather(ubuf, [lane, usub, col])
            iv = plsc.load_gather(ibuf, [lane, isub, col])
            acc = acc + uv * iv
        outv[sl] = acc
        return 0

    lax.fori_loop(0, CHUNKS_PER_W, chunk_step, 0)

    pltpu.sync_copy(outv, out_hbm.at[pl.ds(base, B_PER_W)])


@jax.jit
def kernel(user_ids, item_ids, user_table, item_table, user_bias_table,
           item_bias_table, global_bias):
    uid32 = user_ids.astype(jnp.int32)
    iid32 = item_ids.astype(jnp.int32)
    gb16 = jnp.broadcast_to(global_bias, (LANES,))
    ubg = jnp.take(user_bias_table, uid32, axis=0).reshape(BATCH)
    ibg = jnp.take(item_bias_table, iid32, axis=0).reshape(BATCH)
    num_users, num_items = user_table.shape[0], item_table.shape[0]

    mesh = plsc.VectorSubcoreMesh(core_axis_name="c", subcore_axis_name="s",
                                  num_cores=NUM_CORES,
                                  num_subcores=NUM_SUBCORES)
    mf = pl.kernel(
        _mf_body,
        out_type=jax.ShapeDtypeStruct((BATCH,), jnp.float32),
        mesh=mesh,
        scratch_types=[
            pltpu.VMEM((B_PER_W,), jnp.int32),
            pltpu.VMEM((B_PER_W,), jnp.int32),
            pltpu.VMEM((CHUNK, SUBROWS, EMBED_DIM), jnp.float32),
            pltpu.VMEM((CHUNK, SUBROWS, EMBED_DIM), jnp.float32),
            pltpu.VMEM((B_PER_W,), jnp.float32),
            pltpu.VMEM((B_PER_W,), jnp.float32),
            pltpu.VMEM((LANES,), jnp.float32),
            pltpu.VMEM((B_PER_W,), jnp.float32),
            pltpu.SemaphoreType.DMA,
            pltpu.SemaphoreType.DMA,
        ],
        compiler_params=_PARAMS,
    )
    return mf(uid32, iid32,
              user_table.reshape(num_users // SUBROWS, SUBROWS, EMBED_DIM),
              item_table.reshape(num_items // SUBROWS, SUBROWS, EMBED_DIM),
              ubg, ibg, gb16)
